# Initial kernel scaffold; baseline (speedup 1.0000x reference)
import jax
import jax.numpy as jnp
from jax.experimental import pallas as pl


def kernel(x, edge_index, W_heads, a_src_heads, a_dst_heads, W2, a2_src, a2_dst, Wih, Whh, b_lstm, Wl, bl):
    raise NotImplementedError("write your pallas kernel here")


# speedup vs baseline: 1.0002x; 1.0002x over previous
import jax
import jax.numpy as jnp
from jax.experimental import pallas as pl

N = 10000
E = 320000
NHEADS = 4
ALPHA = 0.2


def _gat_layer(h, src, dst, W, a_s, a_d):
    Wh = h @ W
    f_s = Wh @ a_s
    f_d = Wh @ a_d
    e = jax.nn.leaky_relu(f_s[src] + f_d[dst], negative_slope=ALPHA)
    m = jax.ops.segment_max(e, dst, num_segments=N)
    m = jax.lax.stop_gradient(jnp.where(jnp.isfinite(m), m, 0.0))
    ex = jnp.exp(e - m[dst])
    denom = jax.ops.segment_sum(ex, dst, num_segments=N)
    coef = ex / (denom[dst] + 1e-16)
    out = jax.ops.segment_sum(coef[:, None] * Wh[src], dst, num_segments=N)
    return out


def _head_kernel(hf_ref, wl_ref, bl_ref, out_ref):
    logits = hf_ref[...] @ wl_ref[...] + bl_ref[...]
    mx = jnp.max(logits, axis=1, keepdims=True)
    lse = jnp.log(jnp.sum(jnp.exp(logits - mx), axis=1, keepdims=True)) + mx
    out_ref[...] = logits - lse


def kernel(x, edge_index, W_heads, a_src_heads, a_dst_heads, W2, a2_src, a2_dst, Wih, Whh, b_lstm, Wl, bl):
    src = edge_index[0]
    dst = edge_index[1]
    heads = [jax.nn.elu(_gat_layer(x, src, dst, W_heads[i], a_src_heads[i], a_dst_heads[i]))
             for i in range(NHEADS)]
    h = jnp.concatenate(heads, axis=1)
    h = jax.nn.elu(_gat_layer(h, src, dst, W2, a2_src, a2_dst))

    def step(carry, xt):
        hh, cc = carry
        gates = xt @ Wih + hh @ Whh + b_lstm
        i, f, g, o = jnp.split(gates, 4)
        cc = jax.nn.sigmoid(f) * cc + jax.nn.sigmoid(i) * jnp.tanh(g)
        hh = jax.nn.sigmoid(o) * jnp.tanh(cc)
        return (hh, cc), None

    init = (jnp.zeros((h.shape[1],), dtype=h.dtype), jnp.zeros((h.shape[1],), dtype=h.dtype))
    (hf, _), _ = jax.lax.scan(step, init, h)

    out = pl.pallas_call(
        _head_kernel,
        out_shape=jax.ShapeDtypeStruct((1, Wl.shape[1]), jnp.float32),
    )(hf[None, :], Wl, bl[None, :])
    return out


# speedup vs baseline: 1.8071x; 1.8067x over previous
import jax
import jax.numpy as jnp
from jax.experimental import pallas as pl
from jax.experimental.pallas import tpu as pltpu

N = 10000
E = 320000
NHEADS = 4
ALPHA = 0.2
LIN = 64


def _gat_layer(h, src, dst, W, a_s, a_d):
    Wh = h @ W
    f_s = Wh @ a_s
    f_d = Wh @ a_d
    e = jax.nn.leaky_relu(f_s[src] + f_d[dst], negative_slope=ALPHA)
    ex = jnp.exp(e)
    denom = jax.ops.segment_sum(ex, dst, num_segments=N)
    num = jax.ops.segment_sum(ex[:, None] * Wh[src], dst, num_segments=N)
    return num / (denom[:, None] + 1e-16)


def _lstm_kernel(h_ref, wih_ref, whh_ref, b_ref, wl_ref, bl_ref, out_ref, pre_ref):
    pre_ref[...] = (
        jnp.dot(h_ref[...], wih_ref[...], preferred_element_type=jnp.float32)
        + b_ref[...]
    )
    whh = whh_ref[...]

    def step(t, carry):
        hh, cc = carry
        gates = pre_ref[pl.ds(t, 1), :] + jnp.dot(
            hh, whh, preferred_element_type=jnp.float32
        )
        i_g = gates[:, 0:LIN]
        f_g = gates[:, LIN:2 * LIN]
        g_g = gates[:, 2 * LIN:3 * LIN]
        o_g = gates[:, 3 * LIN:4 * LIN]
        cc = jax.nn.sigmoid(f_g) * cc + jax.nn.sigmoid(i_g) * jnp.tanh(g_g)
        hh = jax.nn.sigmoid(o_g) * jnp.tanh(cc)
        return (hh, cc)

    init = (jnp.zeros((1, LIN), jnp.float32), jnp.zeros((1, LIN), jnp.float32))
    hf, _ = jax.lax.fori_loop(0, N, step, init)
    logits = jnp.dot(hf, wl_ref[...], preferred_element_type=jnp.float32) + bl_ref[...]
    mx = jnp.max(logits, axis=1, keepdims=True)
    lse = jnp.log(jnp.sum(jnp.exp(logits - mx), axis=1, keepdims=True)) + mx
    out_ref[...] = logits - lse


def kernel(x, edge_index, W_heads, a_src_heads, a_dst_heads, W2, a2_src, a2_dst, Wih, Whh, b_lstm, Wl, bl):
    src = edge_index[0].astype(jnp.int32)
    dst = edge_index[1].astype(jnp.int32)
    heads = [jax.nn.elu(_gat_layer(x, src, dst, W_heads[i], a_src_heads[i], a_dst_heads[i]))
             for i in range(NHEADS)]
    h = jnp.concatenate(heads, axis=1)
    h = jax.nn.elu(_gat_layer(h, src, dst, W2, a2_src, a2_dst))

    out = pl.pallas_call(
        _lstm_kernel,
        out_shape=jax.ShapeDtypeStruct((1, Wl.shape[1]), jnp.float32),
        scratch_shapes=[pltpu.VMEM((N, 4 * LIN), jnp.float32)],
    )(h, Wih, Whh, b_lstm[None, :], Wl, bl[None, :])
    return out


# speedup vs baseline: 9.9614x; 5.5125x over previous
import functools

import jax
import jax.numpy as jnp
from jax import lax
from jax.experimental import pallas as pl
from jax.experimental.pallas import tpu as pltpu
from jax.experimental.pallas import tpu_sc as plsc

N = 10000
E = 320000
NFEAT = 128
NHID = 64
NHEADS = 4
LIN = 64
ALPHA = 0.2
EPS = 1e-16

ACCW = 128
TBLW = 128
CHUNK = 128
NC, NS = 2, 16
NW = NC * NS
EPAD = 323584
CHUNKS_PER_TILE = EPAD // (CHUNK * NW)
NPAD = 10112
ROWS_PER_TILE = NPAD // NS


def _proj1_body(x_ref, wc_ref, a_ref, wh_ref, f_ref):
    wh = jnp.dot(x_ref[...], wc_ref[...], preferred_element_type=jnp.float32)
    wh_ref[...] = wh
    f_ref[...] = jnp.dot(wh, a_ref[...], preferred_element_type=jnp.float32)


def _proj1(x, Wcat, A):
    bn = 1000
    grid = N // bn
    return pl.pallas_call(
        _proj1_body,
        grid=(grid,),
        in_specs=[
            pl.BlockSpec((bn, NFEAT), lambda i: (i, 0)),
            pl.BlockSpec((NFEAT, NHEADS * NHID), lambda i: (0, 0)),
            pl.BlockSpec((NHEADS * NHID, 128), lambda i: (0, 0)),
        ],
        out_specs=[
            pl.BlockSpec((bn, NHEADS * NHID), lambda i: (i, 0)),
            pl.BlockSpec((bn, 128), lambda i: (i, 0)),
        ],
        out_shape=[
            jax.ShapeDtypeStruct((N, NHEADS * NHID), jnp.float32),
            jax.ShapeDtypeStruct((N, 128), jnp.float32),
        ],
    )(x, Wcat, A)


def _edge_body(nheads, src_ref, dst_ref, z_ref, whx_refs, fdw_refs,
               out_ref, sidx, didx, rows, fdr, stage, acc, sem):
    cid = lax.axis_index("c")
    sid = lax.axis_index("s")
    wid = sid * NC + cid

    for h in range(nheads):
        whx_hbm = whx_refs[h]
        fdw_hbm = fdw_refs[h]
        pltpu.sync_copy(z_ref, acc.at[pl.ds(sid * ROWS_PER_TILE, ROWS_PER_TILE)])
        pltpu.sync_copy(z_ref.at[pl.ds(0, CHUNK)], stage)
        plsc.subcore_barrier()

        def chunk_body(k, _):
            base = (wid + NW * k) * CHUNK
            pltpu.sync_copy(src_ref.at[pl.ds(base, CHUNK)], sidx)
            pltpu.sync_copy(dst_ref.at[pl.ds(base, CHUNK)], didx)
            pltpu.async_copy(whx_hbm.at[sidx], rows, sem).wait()
            pltpu.async_copy(fdw_hbm.at[didx], fdr, sem).wait()

            def edge(er, _):
                e = rows[er, pl.ds(64, 16)] + fdr[er, pl.ds(0, 16)]
                e = jnp.where(e >= 0.0, e, ALPHA * e)
                ex = jnp.exp(e)
                stage[er, pl.ds(64, 16)] = ex
                for cb in range(4):
                    stage[er, pl.ds(cb * 16, 16)] = (
                        rows[er, pl.ds(cb * 16, 16)] * ex)
                return 0
            lax.fori_loop(0, CHUNK, edge, 0)

            pltpu.sync_copy(stage, acc.at[didx], add=True)
            return 0
        lax.fori_loop(0, CHUNKS_PER_TILE, chunk_body, 0)
        plsc.subcore_barrier()
        row0 = (h * NC + cid) * NPAD + sid * ROWS_PER_TILE
        pltpu.sync_copy(acc.at[pl.ds(sid * ROWS_PER_TILE, ROWS_PER_TILE)],
                        out_ref.at[pl.ds(row0, ROWS_PER_TILE)])
        plsc.subcore_barrier()


@functools.lru_cache(maxsize=None)
def _make_edge_kernel(nheads):
    mesh = plsc.VectorSubcoreMesh(core_axis_name="c", subcore_axis_name="s")

    def body(*refs):
        src_ref, dst_ref, z_ref = refs[:3]
        whx_refs = refs[3:3 + nheads]
        fdw_refs = refs[3 + nheads:3 + 2 * nheads]
        out_ref = refs[3 + 2 * nheads]
        scratch = refs[4 + 2 * nheads:]
        _edge_body(nheads, src_ref, dst_ref, z_ref, whx_refs, fdw_refs,
                   out_ref, *scratch)

    return pl.kernel(
        body,
        out_type=jax.ShapeDtypeStruct((nheads * 2 * NPAD, ACCW), jnp.float32),
        mesh=mesh,
        scratch_types=[
            pltpu.VMEM((CHUNK,), jnp.int32),
            pltpu.VMEM((CHUNK,), jnp.int32),
            pltpu.VMEM((CHUNK, TBLW), jnp.float32),
            pltpu.VMEM((CHUNK, TBLW), jnp.float32),
            pltpu.VMEM((CHUNK, ACCW), jnp.float32),
            pltpu.VMEM_SHARED((NPAD, ACCW), jnp.float32),
            pltpu.SemaphoreType.DMA,
        ],
    )


def _comb1_body(p_ref, w2_ref, a2_ref, wh2_ref, f2_ref):
    cols = []
    for h in range(NHEADS):
        s = p_ref[2 * h] + p_ref[2 * h + 1]
        num = s[:, 0:NHID]
        den = s[:, NHID:NHID + 1]
        v = num / (den + EPS)
        cols.append(jnp.where(v > 0.0, v, jnp.exp(jnp.minimum(v, 0.0)) - 1.0))
    hcat = jnp.concatenate(cols, axis=1)
    wh2 = jnp.dot(hcat, w2_ref[...], preferred_element_type=jnp.float32)
    wh2_ref[...] = wh2
    f2_ref[...] = jnp.dot(wh2, a2_ref[...], preferred_element_type=jnp.float32)


def _comb1(part1, W2, A2):
    bn = ROWS_PER_TILE
    grid = NPAD // bn
    return pl.pallas_call(
        _comb1_body,
        grid=(grid,),
        in_specs=[
            pl.BlockSpec((2 * NHEADS, bn, ACCW), lambda i: (0, i, 0)),
            pl.BlockSpec((NHEADS * NHID, LIN), lambda i: (0, 0)),
            pl.BlockSpec((LIN, 128), lambda i: (0, 0)),
        ],
        out_specs=[
            pl.BlockSpec((bn, LIN), lambda i: (i, 0)),
            pl.BlockSpec((bn, 128), lambda i: (i, 0)),
        ],
        out_shape=[
            jax.ShapeDtypeStruct((NPAD, LIN), jnp.float32),
            jax.ShapeDtypeStruct((NPAD, 128), jnp.float32),
        ],
    )(part1, W2, A2)


def _sig(x):
    return 1.0 / (1.0 + jnp.exp(-x))


def _lstm_body(p_ref, wih_ref, whh_ref, b_ref, wl_ref, bl_ref, out_ref, pre_ref):
    s = p_ref[0] + p_ref[1]
    num = s[:, 0:LIN]
    den = s[:, LIN:LIN + 1]
    v = num / (den + EPS)
    h2 = jnp.where(v > 0.0, v, jnp.exp(jnp.minimum(v, 0.0)) - 1.0)
    pre_ref[...] = (
        jnp.dot(h2, wih_ref[...], preferred_element_type=jnp.float32) + b_ref[...]
    )
    whh = whh_ref[...]

    def step(t, carry):
        hh, cc = carry
        gates = pre_ref[pl.ds(t, 1), :] + jnp.dot(
            hh, whh, preferred_element_type=jnp.float32
        )
        i_g = gates[:, 0:LIN]
        f_g = gates[:, LIN:2 * LIN]
        g_g = gates[:, 2 * LIN:3 * LIN]
        o_g = gates[:, 3 * LIN:4 * LIN]
        cc = _sig(f_g) * cc + _sig(i_g) * jnp.tanh(g_g)
        hh = _sig(o_g) * jnp.tanh(cc)
        return (hh, cc)

    init = (jnp.zeros((1, LIN), jnp.float32), jnp.zeros((1, LIN), jnp.float32))
    hf, _ = lax.fori_loop(0, N, step, init)
    logits = jnp.dot(hf, wl_ref[...], preferred_element_type=jnp.float32) + bl_ref[...]
    mx = jnp.max(logits, axis=1, keepdims=True)
    lse = jnp.log(jnp.sum(jnp.exp(logits - mx), axis=1, keepdims=True)) + mx
    out_ref[...] = logits - lse


def _lstm(part2, Wih, Whh, b_lstm, Wl, bl):
    return pl.pallas_call(
        _lstm_body,
        out_shape=jax.ShapeDtypeStruct((1, Wl.shape[1]), jnp.float32),
        scratch_shapes=[pltpu.VMEM((NPAD, 4 * LIN), jnp.float32)],
    )(part2, Wih, Whh, b_lstm[None, :], Wl, bl[None, :])


def kernel(x, edge_index, W_heads, a_src_heads, a_dst_heads, W2, a2_src, a2_dst,
           Wih, Whh, b_lstm, Wl, bl):
    src = jnp.full((EPAD,), N, jnp.int32).at[:E].set(edge_index[0].astype(jnp.int32))
    dst = jnp.full((EPAD,), N, jnp.int32).at[:E].set(edge_index[1].astype(jnp.int32))

    Wcat = jnp.transpose(W_heads, (1, 0, 2)).reshape(NFEAT, NHEADS * NHID)
    A = jnp.zeros((NHEADS * NHID, 128), jnp.float32)
    for h in range(NHEADS):
        A = A.at[h * NHID:(h + 1) * NHID, h].set(a_src_heads[h])
        A = A.at[h * NHID:(h + 1) * NHID, NHEADS + h].set(a_dst_heads[h])
    A2 = jnp.zeros((LIN, 128), jnp.float32)
    A2 = A2.at[:, 0].set(a2_src)
    A2 = A2.at[:, 1].set(a2_dst)
    zeros_hbm = jnp.zeros((ROWS_PER_TILE, ACCW), jnp.float32)

    wh1, f1 = _proj1(x, Wcat, A)

    whx1, fdw1 = [], []
    for h in range(NHEADS):
        wh_h = jnp.zeros((NPAD, NHID), jnp.float32).at[:N].set(
            wh1[:, h * NHID:(h + 1) * NHID])
        fs_h = jnp.zeros((NPAD,), jnp.float32).at[:N].set(f1[:, h])
        fd_h = jnp.zeros((NPAD,), jnp.float32).at[:N].set(f1[:, NHEADS + h])
        whx1.append(jnp.concatenate(
            [wh_h, jnp.broadcast_to(fs_h[:, None], (NPAD, TBLW - NHID))],
            axis=1))
        fdw1.append(jnp.broadcast_to(fd_h[:, None], (NPAD, TBLW)))

    part1 = _make_edge_kernel(NHEADS)(src, dst, zeros_hbm, *whx1, *fdw1)
    part1 = part1.reshape(2 * NHEADS, NPAD, ACCW)

    wh2, f2 = _comb1(part1, W2, A2)
    whx2 = jnp.concatenate(
        [wh2, jnp.broadcast_to(f2[:, 0:1], (NPAD, TBLW - LIN))], axis=1)
    fdw2 = jnp.broadcast_to(f2[:, 1:2], (NPAD, TBLW))

    part2 = _make_edge_kernel(1)(src, dst, zeros_hbm, whx2, fdw2)
    part2 = part2.reshape(2, NPAD, ACCW)

    return _lstm(part2, Wih, Whh, b_lstm, Wl, bl)


# speedup vs baseline: 11.0290x; 1.1072x over previous
import functools

import jax
import jax.numpy as jnp
from jax import lax
from jax.experimental import pallas as pl
from jax.experimental.pallas import tpu as pltpu
from jax.experimental.pallas import tpu_sc as plsc

N = 10000
E = 320000
NFEAT = 128
NHID = 64
NHEADS = 4
LIN = 64
ALPHA = 0.2
EPS = 1e-16

ACCW = 128
TBLW = 128
CHUNK = 64
NC, NS = 2, 16
NW = NC * NS
EPAD = 325632
CHUNKS_PER_TILE = EPAD // (CHUNK * NW)
NPAD = 10112
ROWS_PER_TILE = NPAD // NS


def _proj1_body(x_ref, wc_ref, a_ref, wh_ref, f_ref):
    wh = jnp.dot(x_ref[...], wc_ref[...], preferred_element_type=jnp.float32)
    wh_ref[...] = wh
    f_ref[...] = jnp.dot(wh, a_ref[...], preferred_element_type=jnp.float32)


def _proj1(x, Wcat, A):
    bn = 1000
    grid = N // bn
    return pl.pallas_call(
        _proj1_body,
        grid=(grid,),
        in_specs=[
            pl.BlockSpec((bn, NFEAT), lambda i: (i, 0)),
            pl.BlockSpec((NFEAT, NHEADS * NHID), lambda i: (0, 0)),
            pl.BlockSpec((NHEADS * NHID, 128), lambda i: (0, 0)),
        ],
        out_specs=[
            pl.BlockSpec((bn, NHEADS * NHID), lambda i: (i, 0)),
            pl.BlockSpec((bn, 128), lambda i: (i, 0)),
        ],
        out_shape=[
            jax.ShapeDtypeStruct((N, NHEADS * NHID), jnp.float32),
            jax.ShapeDtypeStruct((N, 128), jnp.float32),
        ],
    )(x, Wcat, A)


def _edge_body(nheads, src_ref, dst_ref, z_ref, whx_refs, fdw_refs, out_ref,
               sidx0, sidx1, didx0, didx1, rows0, rows1, fdr0, fdr1,
               stage, acc, sem0, sem1):
    cid = lax.axis_index("c")
    sid = lax.axis_index("s")
    wid = sid * NC + cid
    SIDX, DIDX = (sidx0, sidx1), (didx0, didx1)
    ROWS, FDR, SEM = (rows0, rows1), (fdr0, fdr1), (sem0, sem1)

    for h in range(nheads):
        whx_hbm = whx_refs[h]
        fdw_hbm = fdw_refs[h]

        def load_and_start(b, k):
            base = (wid + NW * k) * CHUNK
            pltpu.sync_copy(src_ref.at[pl.ds(base, CHUNK)], SIDX[b])
            pltpu.sync_copy(dst_ref.at[pl.ds(base, CHUNK)], DIDX[b])
            pltpu.async_copy(whx_hbm.at[SIDX[b]], ROWS[b], SEM[b])
            pltpu.async_copy(fdw_hbm.at[DIDX[b]], FDR[b], SEM[b])

        def drain(b):
            pltpu.make_async_copy(
                whx_hbm.at[pl.ds(0, CHUNK)], ROWS[b], SEM[b]).wait()
            pltpu.make_async_copy(
                fdw_hbm.at[pl.ds(0, CHUNK)], FDR[b], SEM[b]).wait()

        def compute_scatter(b):
            rows, fdr = ROWS[b], FDR[b]

            def edge(er, _):
                e = rows[er, pl.ds(64, 16)] + fdr[er, pl.ds(0, 16)]
                e = jnp.where(e >= 0.0, e, ALPHA * e)
                ex = jnp.exp(e)
                stage[er, pl.ds(64, 16)] = ex
                for cb in range(4):
                    stage[er, pl.ds(cb * 16, 16)] = (
                        rows[er, pl.ds(cb * 16, 16)] * ex)
                return 0
            lax.fori_loop(0, CHUNK, edge, 0)
            pltpu.sync_copy(stage, acc.at[DIDX[b]], add=True)

        pltpu.sync_copy(z_ref, acc.at[pl.ds(sid * ROWS_PER_TILE, ROWS_PER_TILE)])
        pltpu.sync_copy(z_ref.at[pl.ds(0, CHUNK)], stage)
        plsc.subcore_barrier()

        load_and_start(0, jnp.int32(0))

        def pair_body(g, _):
            for b in range(2):
                k = 2 * g + b
                load_and_start(1 - b, k + 1)
                drain(b)
                compute_scatter(b)
            return 0
        lax.fori_loop(0, (CHUNKS_PER_TILE - 1) // 2, pair_body, 0)
        drain(0)
        compute_scatter(0)

        plsc.subcore_barrier()
        row0 = (h * NC + cid) * NPAD + sid * ROWS_PER_TILE
        pltpu.sync_copy(acc.at[pl.ds(sid * ROWS_PER_TILE, ROWS_PER_TILE)],
                        out_ref.at[pl.ds(row0, ROWS_PER_TILE)])
        plsc.subcore_barrier()


@functools.lru_cache(maxsize=None)
def _make_edge_kernel(nheads):
    mesh = plsc.VectorSubcoreMesh(core_axis_name="c", subcore_axis_name="s")

    def body(*refs):
        src_ref, dst_ref, z_ref = refs[:3]
        whx_refs = refs[3:3 + nheads]
        fdw_refs = refs[3 + nheads:3 + 2 * nheads]
        out_ref = refs[3 + 2 * nheads]
        scratch = refs[4 + 2 * nheads:]
        _edge_body(nheads, src_ref, dst_ref, z_ref, whx_refs, fdw_refs,
                   out_ref, *scratch)

    return pl.kernel(
        body,
        out_type=jax.ShapeDtypeStruct((nheads * 2 * NPAD, ACCW), jnp.float32),
        mesh=mesh,
        scratch_types=[
            pltpu.VMEM((CHUNK,), jnp.int32),
            pltpu.VMEM((CHUNK,), jnp.int32),
            pltpu.VMEM((CHUNK,), jnp.int32),
            pltpu.VMEM((CHUNK,), jnp.int32),
            pltpu.VMEM((CHUNK, TBLW), jnp.float32),
            pltpu.VMEM((CHUNK, TBLW), jnp.float32),
            pltpu.VMEM((CHUNK, TBLW), jnp.float32),
            pltpu.VMEM((CHUNK, TBLW), jnp.float32),
            pltpu.VMEM((CHUNK, ACCW), jnp.float32),
            pltpu.VMEM_SHARED((NPAD, ACCW), jnp.float32),
            pltpu.SemaphoreType.DMA,
            pltpu.SemaphoreType.DMA,
        ],
    )


def _comb1_body(p_ref, w2_ref, a2_ref, wh2_ref, f2_ref):
    cols = []
    for h in range(NHEADS):
        s = p_ref[2 * h] + p_ref[2 * h + 1]
        num = s[:, 0:NHID]
        den = s[:, NHID:NHID + 1]
        v = num / (den + EPS)
        cols.append(jnp.where(v > 0.0, v, jnp.exp(jnp.minimum(v, 0.0)) - 1.0))
    hcat = jnp.concatenate(cols, axis=1)
    wh2 = jnp.dot(hcat, w2_ref[...], preferred_element_type=jnp.float32)
    wh2_ref[...] = wh2
    f2_ref[...] = jnp.dot(wh2, a2_ref[...], preferred_element_type=jnp.float32)


def _comb1(part1, W2, A2):
    bn = ROWS_PER_TILE
    grid = NPAD // bn
    return pl.pallas_call(
        _comb1_body,
        grid=(grid,),
        in_specs=[
            pl.BlockSpec((2 * NHEADS, bn, ACCW), lambda i: (0, i, 0)),
            pl.BlockSpec((NHEADS * NHID, LIN), lambda i: (0, 0)),
            pl.BlockSpec((LIN, 128), lambda i: (0, 0)),
        ],
        out_specs=[
            pl.BlockSpec((bn, LIN), lambda i: (i, 0)),
            pl.BlockSpec((bn, 128), lambda i: (i, 0)),
        ],
        out_shape=[
            jax.ShapeDtypeStruct((NPAD, LIN), jnp.float32),
            jax.ShapeDtypeStruct((NPAD, 128), jnp.float32),
        ],
    )(part1, W2, A2)


def _sig(x):
    return 1.0 / (1.0 + jnp.exp(-x))


def _lstm_body(p_ref, wih_ref, whh_ref, b_ref, wl_ref, bl_ref, out_ref, pre_ref):
    s = p_ref[0] + p_ref[1]
    num = s[:, 0:LIN]
    den = s[:, LIN:LIN + 1]
    v = num / (den + EPS)
    h2 = jnp.where(v > 0.0, v, jnp.exp(jnp.minimum(v, 0.0)) - 1.0)
    pre_ref[...] = (
        jnp.dot(h2, wih_ref[...], preferred_element_type=jnp.float32) + b_ref[...]
    )
    whh = whh_ref[...]

    def step(t, carry):
        hh, cc = carry
        gates = pre_ref[pl.ds(t, 1), :] + jnp.dot(
            hh, whh, preferred_element_type=jnp.float32
        )
        i_g = gates[:, 0:LIN]
        f_g = gates[:, LIN:2 * LIN]
        g_g = gates[:, 2 * LIN:3 * LIN]
        o_g = gates[:, 3 * LIN:4 * LIN]
        cc = _sig(f_g) * cc + _sig(i_g) * jnp.tanh(g_g)
        hh = _sig(o_g) * jnp.tanh(cc)
        return (hh, cc)

    init = (jnp.zeros((1, LIN), jnp.float32), jnp.zeros((1, LIN), jnp.float32))
    hf, _ = lax.fori_loop(0, N, step, init)
    logits = jnp.dot(hf, wl_ref[...], preferred_element_type=jnp.float32) + bl_ref[...]
    mx = jnp.max(logits, axis=1, keepdims=True)
    lse = jnp.log(jnp.sum(jnp.exp(logits - mx), axis=1, keepdims=True)) + mx
    out_ref[...] = logits - lse


def _lstm(part2, Wih, Whh, b_lstm, Wl, bl):
    return pl.pallas_call(
        _lstm_body,
        out_shape=jax.ShapeDtypeStruct((1, Wl.shape[1]), jnp.float32),
        scratch_shapes=[pltpu.VMEM((NPAD, 4 * LIN), jnp.float32)],
    )(part2, Wih, Whh, b_lstm[None, :], Wl, bl[None, :])


def kernel(x, edge_index, W_heads, a_src_heads, a_dst_heads, W2, a2_src, a2_dst,
           Wih, Whh, b_lstm, Wl, bl):
    src = jnp.full((EPAD,), N, jnp.int32).at[:E].set(edge_index[0].astype(jnp.int32))
    dst = jnp.full((EPAD,), N, jnp.int32).at[:E].set(edge_index[1].astype(jnp.int32))

    Wcat = jnp.transpose(W_heads, (1, 0, 2)).reshape(NFEAT, NHEADS * NHID)
    A = jnp.zeros((NHEADS * NHID, 128), jnp.float32)
    for h in range(NHEADS):
        A = A.at[h * NHID:(h + 1) * NHID, h].set(a_src_heads[h])
        A = A.at[h * NHID:(h + 1) * NHID, NHEADS + h].set(a_dst_heads[h])
    A2 = jnp.zeros((LIN, 128), jnp.float32)
    A2 = A2.at[:, 0].set(a2_src)
    A2 = A2.at[:, 1].set(a2_dst)
    zeros_hbm = jnp.zeros((ROWS_PER_TILE, ACCW), jnp.float32)

    wh1, f1 = _proj1(x, Wcat, A)

    whx1, fdw1 = [], []
    for h in range(NHEADS):
        wh_h = jnp.zeros((NPAD, NHID), jnp.float32).at[:N].set(
            wh1[:, h * NHID:(h + 1) * NHID])
        fs_h = jnp.zeros((NPAD,), jnp.float32).at[:N].set(f1[:, h])
        fd_h = jnp.zeros((NPAD,), jnp.float32).at[:N].set(f1[:, NHEADS + h])
        whx1.append(jnp.concatenate(
            [wh_h, jnp.broadcast_to(fs_h[:, None], (NPAD, TBLW - NHID))],
            axis=1))
        fdw1.append(jnp.broadcast_to(fd_h[:, None], (NPAD, TBLW)))

    part1 = _make_edge_kernel(NHEADS)(src, dst, zeros_hbm, *whx1, *fdw1)
    part1 = part1.reshape(2 * NHEADS, NPAD, ACCW)

    wh2, f2 = _comb1(part1, W2, A2)
    whx2 = jnp.concatenate(
        [wh2, jnp.broadcast_to(f2[:, 0:1], (NPAD, TBLW - LIN))], axis=1)
    fdw2 = jnp.broadcast_to(f2[:, 1:2], (NPAD, TBLW))

    part2 = _make_edge_kernel(1)(src, dst, zeros_hbm, whx2, fdw2)
    part2 = part2.reshape(2, NPAD, ACCW)

    return _lstm(part2, Wih, Whh, b_lstm, Wl, bl)
